# baseline (device time: 55707 ns/iter reference)
import jax
import jax.numpy as jnp
from jax import lax
from jax.experimental import pallas as pl
from jax.experimental.pallas import tpu as pltpu

N_DEV = 8
ORD = ((0, 1, 2), (1, 2, 0), (2, 0, 1))
SLOT_BASE = (0, 4, 6)
N_TILE = 4


def kernel(x, dy):
    m, d = x.shape
    _, f = dy.shape
    ch = d // N_DEV
    widths = (11 * ch, 11 * ch, 10 * ch)
    col0s = (0, widths[0], widths[0] + widths[1])
    tmax = 3 * ch

    x_bf = x.astype(jnp.bfloat16)

    def body(x_ref, dy_hbm, out_ref, acc, dy_bf, stage,
             recv0, recv1, recv2,
             copy_sems, ss0, rs0, ss1, rs1, ss2, rs2):
        recvs = (recv0, recv1, recv2)
        sss = (ss0, ss1, ss2)
        rss = (rs0, rs1, rs2)

        j = lax.axis_index("i")
        jj = j % 4
        c = [((jj == 1) | (jj == 2)).astype(jnp.int32),
             (jj >= 2).astype(jnp.int32),
             j // 4]

        def qid(qx, qy, qz):
            return qz * 4 + qy * 2 + (qx + qy) % 2

        partners = []
        for dd in range(3):
            p = list(c)
            p[dd] = 1 - p[dd]
            partners.append(qid(p[0], p[1], p[2]))

        def chunk(g, k, i, sent):
            d0, d1, d2 = ORD[g]
            bits = [None, None, None]
            if k == 0:
                bits[d0] = 1 - c[d0] if sent else c[d0]
                bits[d1] = (i >> 1) & 1
                bits[d2] = i & 1
            elif k == 1:
                bits[d0] = c[d0]
                bits[d1] = 1 - c[d1] if sent else c[d1]
                bits[d2] = i
            else:
                bits[d0] = c[d0]
                bits[d1] = c[d1]
                bits[d2] = 1 - c[d2] if sent else c[d2]
            return qid(bits[0], bits[1], bits[2])

        tiles = []
        for g in range(3):
            off = 0
            while off < widths[g]:
                tw = min(tmax, widths[g] - off)
                tiles.append((g, col0s[g] + off, tw))
                off += tw

        def tile_copy(idx, slot):
            _, c0, tw = tiles[idx]
            return pltpu.make_async_copy(
                dy_hbm.at[:, pl.ds(c0, tw)],
                stage.at[slot, :, pl.ds(0, tw)],
                copy_sems.at[slot],
            )

        copies = {}
        for idx in (0, 1):
            copies[idx] = tile_copy(idx, idx)
            copies[idx].start()

        barrier = pltpu.get_barrier_semaphore()
        for dd in range(3):
            pl.semaphore_signal(barrier, inc=1, device_id=(partners[dd],),
                                device_id_type=pl.DeviceIdType.MESH)
        pl.semaphore_wait(barrier, 3)

        def partial(q, g):
            return lax.dot_general(
                x_ref[:, pl.ds(q * ch, ch)],
                dy_bf[:, pl.ds(col0s[g], widths[g])],
                (((0,), (0,)), ((), ())),
                preferred_element_type=jnp.float32,
            ).astype(jnp.bfloat16)

        def make_rdma(g, k, i):
            q = chunk(g, k, i, sent=True)
            slot = SLOT_BASE[k] + i
            return pltpu.make_async_remote_copy(
                src_ref=acc.at[pl.ds(q * ch, ch),
                               pl.ds(col0s[g], widths[g])],
                dst_ref=recvs[g].at[slot],
                send_sem=sss[g].at[slot],
                recv_sem=rss[g].at[slot],
                device_id=(partners[ORD[g][k]],),
                device_id_type=pl.DeviceIdType.MESH,
            )

        inflight = {}

        for idx, (g, c0, tw) in enumerate(tiles):
            slot = idx % 2
            copies[idx].wait()
            dy_bf[:, pl.ds(c0, tw)] = stage[slot, :, :tw].astype(jnp.bfloat16)
            if idx + 2 < len(tiles):
                copies[idx + 2] = tile_copy(idx + 2, slot)
                copies[idx + 2].start()
            if (idx + 1) % N_TILE == 0:
                for i in range(4):
                    q = chunk(g, 0, i, sent=True)
                    acc[pl.ds(q * ch, ch),
                        pl.ds(col0s[g], widths[g])] = partial(q, g)
                    rdma = make_rdma(g, 0, i)
                    rdma.start()
                    inflight[(g, 0, i)] = rdma
        for i in range(4):
            for g in range(3):
                q = chunk(g, 0, i, sent=False)
                acc[pl.ds(q * ch, ch), pl.ds(col0s[g], widths[g])] = partial(q, g)

        for k in range(3):
            for g in range(3):
                for i in range(4 >> k):
                    slot = SLOT_BASE[k] + i
                    inflight[(g, k, i)].wait_recv()
                    q = chunk(g, k, i, sent=False)
                    rows = pl.ds(q * ch, ch)
                    cols = pl.ds(col0s[g], widths[g])
                    summed = (acc[rows, cols].astype(jnp.float32)
                              + recvs[g][slot].astype(jnp.float32))
                    if k < 2:
                        acc[rows, cols] = summed.astype(jnp.bfloat16)
                    else:
                        out_ref[:, cols] = summed
            if k < 2:
                for g in range(3):
                    for i in range(4 >> (k + 1)):
                        rdma = make_rdma(g, k + 1, i)
                        rdma.start()
                        inflight[(g, k + 1, i)] = rdma

        for k in range(3):
            for g in range(3):
                for i in range(4 >> k):
                    inflight[(g, k, i)].wait_send()

    return pl.pallas_call(
        body,
        out_shape=jax.ShapeDtypeStruct((ch, f), jnp.float32),
        in_specs=[
            pl.BlockSpec(memory_space=pltpu.VMEM),
            pl.BlockSpec(memory_space=pl.ANY),
        ],
        out_specs=pl.BlockSpec(memory_space=pltpu.VMEM),
        scratch_shapes=[
            pltpu.VMEM((d, f), jnp.bfloat16),
            pltpu.VMEM((m, f), jnp.bfloat16),
            pltpu.VMEM((2, m, tmax), jnp.float32),
            pltpu.VMEM((7, ch, widths[0]), jnp.bfloat16),
            pltpu.VMEM((7, ch, widths[1]), jnp.bfloat16),
            pltpu.VMEM((7, ch, widths[2]), jnp.bfloat16),
            pltpu.SemaphoreType.DMA((2,)),
            pltpu.SemaphoreType.DMA((7,)),
            pltpu.SemaphoreType.DMA((7,)),
            pltpu.SemaphoreType.DMA((7,)),
            pltpu.SemaphoreType.DMA((7,)),
            pltpu.SemaphoreType.DMA((7,)),
            pltpu.SemaphoreType.DMA((7,)),
        ],
        compiler_params=pltpu.CompilerParams(collective_id=0),
    )(x_bf, dy)


# device time: 48178 ns/iter; 1.1563x vs baseline; 1.1563x over previous
import jax
import jax.numpy as jnp
from jax import lax
from jax.experimental import pallas as pl
from jax.experimental.pallas import tpu as pltpu

N_DEV = 8
ORD = ((0, 1, 2), (1, 2, 0), (2, 0, 1))
SLOT_BASE = (0, 4, 6)


def kernel(x, dy):
    m, d = x.shape
    _, f = dy.shape
    ch = d // N_DEV
    widths = (11 * ch, 11 * ch, 10 * ch)
    col0s = (0, widths[0], widths[0] + widths[1])

    x_bf = x.astype(jnp.bfloat16)
    dy_bf = dy.astype(jnp.bfloat16)

    def body(x_ref, dy_ref, out_ref, acc,
             recv0, recv1, recv2, ss0, rs0, ss1, rs1, ss2, rs2):
        recvs = (recv0, recv1, recv2)
        sss = (ss0, ss1, ss2)
        rss = (rs0, rs1, rs2)

        j = lax.axis_index("i")
        jj = j % 4
        c = [((jj == 1) | (jj == 2)).astype(jnp.int32),
             (jj >= 2).astype(jnp.int32),
             j // 4]

        def qid(qx, qy, qz):
            return qz * 4 + qy * 2 + (qx + qy) % 2

        partners = []
        for dd in range(3):
            p = list(c)
            p[dd] = 1 - p[dd]
            partners.append(qid(p[0], p[1], p[2]))

        def chunk(g, k, i, sent):
            d0, d1, d2 = ORD[g]

            def rel(dd, r):
                return c[dd] if r == 0 else 1 - c[dd]

            bits = [None, None, None]
            if k == 0:
                bits[d0] = rel(d0, 1 if sent else 0)
                bits[d1] = rel(d1, (i >> 1) & 1)
                bits[d2] = rel(d2, i & 1)
            elif k == 1:
                bits[d0] = rel(d0, 0)
                bits[d1] = rel(d1, 1 if sent else 0)
                bits[d2] = rel(d2, i)
            else:
                bits[d0] = rel(d0, 0)
                bits[d1] = rel(d1, 0)
                bits[d2] = rel(d2, 1 if sent else 0)
            return qid(bits[0], bits[1], bits[2])

        barrier = pltpu.get_barrier_semaphore()
        for dd in range(3):
            pl.semaphore_signal(barrier, inc=1, device_id=(partners[dd],),
                                device_id_type=pl.DeviceIdType.MESH)
        pl.semaphore_wait(barrier, 3)

        def partial(q, g):
            return lax.dot_general(
                x_ref[:, pl.ds(q * ch, ch)],
                dy_ref[:, pl.ds(col0s[g], widths[g])],
                (((0,), (0,)), ((), ())),
                preferred_element_type=jnp.float32,
            ).astype(jnp.bfloat16)

        def make_rdma(g, k, i):
            q = chunk(g, k, i, sent=True)
            slot = SLOT_BASE[k] + i
            return pltpu.make_async_remote_copy(
                src_ref=acc.at[pl.ds(q * ch, ch),
                               pl.ds(col0s[g], widths[g])],
                dst_ref=recvs[g].at[slot],
                send_sem=sss[g].at[slot],
                recv_sem=rss[g].at[slot],
                device_id=(partners[ORD[g][k]],),
                device_id_type=pl.DeviceIdType.MESH,
            )

        inflight = {}

        for i in range(4):
            for g in range(3):
                q = chunk(g, 0, i, sent=True)
                acc[pl.ds(q * ch, ch), pl.ds(col0s[g], widths[g])] = partial(q, g)
                rdma = make_rdma(g, 0, i)
                rdma.start()
                inflight[(g, 0, i)] = rdma
        for i in range(4):
            for g in range(3):
                q = chunk(g, 0, i, sent=False)
                acc[pl.ds(q * ch, ch), pl.ds(col0s[g], widths[g])] = partial(q, g)

        def fold(g, k, i):
            slot = SLOT_BASE[k] + i
            inflight[(g, k, i)].wait_recv()
            q = chunk(g, k, i, sent=False)
            rows = pl.ds(q * ch, ch)
            cols = pl.ds(col0s[g], widths[g])
            summed = (acc[rows, cols].astype(jnp.float32)
                      + recvs[g][slot].astype(jnp.float32))
            if k < 2:
                acc[rows, cols] = summed.astype(jnp.bfloat16)
            else:
                out_ref[:, cols] = summed

        for g in range(3):
            for i in (2, 3):
                fold(g, 0, i)
            for i2 in range(2):
                rdma = make_rdma(g, 1, i2)
                rdma.start()
                inflight[(g, 1, i2)] = rdma
        for g in range(3):
            for i in (0, 1):
                fold(g, 0, i)
        for g in range(3):
            fold(g, 1, 1)
            rdma = make_rdma(g, 2, 0)
            rdma.start()
            inflight[(g, 2, 0)] = rdma
        for g in range(3):
            fold(g, 1, 0)
        for g in range(3):
            fold(g, 2, 0)

        for k in range(3):
            for g in range(3):
                for i in range(4 >> k):
                    inflight[(g, k, i)].wait_send()

    return pl.pallas_call(
        body,
        out_shape=jax.ShapeDtypeStruct((ch, f), jnp.float32),
        in_specs=[
            pl.BlockSpec(memory_space=pltpu.VMEM),
            pl.BlockSpec(memory_space=pltpu.VMEM),
        ],
        out_specs=pl.BlockSpec(memory_space=pltpu.VMEM),
        scratch_shapes=[
            pltpu.VMEM((d, f), jnp.bfloat16),
            pltpu.VMEM((7, ch, widths[0]), jnp.bfloat16),
            pltpu.VMEM((7, ch, widths[1]), jnp.bfloat16),
            pltpu.VMEM((7, ch, widths[2]), jnp.bfloat16),
            pltpu.SemaphoreType.DMA((7,)),
            pltpu.SemaphoreType.DMA((7,)),
            pltpu.SemaphoreType.DMA((7,)),
            pltpu.SemaphoreType.DMA((7,)),
            pltpu.SemaphoreType.DMA((7,)),
            pltpu.SemaphoreType.DMA((7,)),
        ],
        compiler_params=pltpu.CompilerParams(collective_id=0),
    )(x_bf, dy_bf)


# device time: 44224 ns/iter; 1.2597x vs baseline; 1.0894x over previous
import jax
import jax.numpy as jnp
from jax import lax
from jax.experimental import pallas as pl
from jax.experimental.pallas import tpu as pltpu

N_DEV = 8
ORD = ((0, 1, 2), (1, 2, 0), (2, 0, 1))
SLOT_BASE = (0, 4, 6)


def kernel(x, dy):
    m, d = x.shape
    _, f = dy.shape
    ch = d // N_DEV
    widths = (11 * ch, 11 * ch, 10 * ch)
    col0s = (0, widths[0], widths[0] + widths[1])

    x_bf = x.astype(jnp.bfloat16)
    dy_bf = dy.astype(jnp.bfloat16)

    def body(x_ref, dy_ref, out_ref, acc,
             recv0, recv1, recv2, ss0, rs0, ss1, rs1, ss2, rs2):
        recvs = (recv0, recv1, recv2)
        sss = (ss0, ss1, ss2)
        rss = (rs0, rs1, rs2)

        j = lax.axis_index("i")
        jj = j % 4
        c = [((jj == 1) | (jj == 2)).astype(jnp.int32),
             (jj >= 2).astype(jnp.int32),
             j // 4]

        def qid(qx, qy, qz):
            return qz * 4 + qy * 2 + (qx + qy) % 2

        partners = []
        for dd in range(3):
            p = list(c)
            p[dd] = 1 - p[dd]
            partners.append(qid(p[0], p[1], p[2]))

        def chunk(g, k, i, sent):
            d0, d1, d2 = ORD[g]

            def rel(dd, r):
                return c[dd] if r == 0 else 1 - c[dd]

            bits = [None, None, None]
            if k == 0:
                bits[d0] = rel(d0, 1 if sent else 0)
                bits[d1] = rel(d1, (i >> 1) & 1)
                bits[d2] = rel(d2, i & 1)
            elif k == 1:
                bits[d0] = rel(d0, 0)
                bits[d1] = rel(d1, 1 if sent else 0)
                bits[d2] = rel(d2, i)
            else:
                bits[d0] = rel(d0, 0)
                bits[d1] = rel(d1, 0)
                bits[d2] = rel(d2, 1 if sent else 0)
            return qid(bits[0], bits[1], bits[2])

        barrier = pltpu.get_barrier_semaphore()
        for dd in range(3):
            pl.semaphore_signal(barrier, inc=1, device_id=(partners[dd],),
                                device_id_type=pl.DeviceIdType.MESH)
        pl.semaphore_wait(barrier, 3)

        def partial(q, g):
            return lax.dot_general(
                x_ref[:, pl.ds(q * ch, ch)],
                dy_ref[:, pl.ds(col0s[g], widths[g])],
                (((0,), (0,)), ((), ())),
                preferred_element_type=jnp.float32,
            ).astype(jnp.bfloat16)

        def make_rdma(g, k, i):
            q = chunk(g, k, i, sent=True)
            slot = SLOT_BASE[k] + i
            return pltpu.make_async_remote_copy(
                src_ref=acc.at[pl.ds(q * ch, ch),
                               pl.ds(col0s[g], widths[g])],
                dst_ref=recvs[g].at[slot],
                send_sem=sss[g].at[slot],
                recv_sem=rss[g].at[slot],
                device_id=(partners[ORD[g][k]],),
                device_id_type=pl.DeviceIdType.MESH,
            )

        inflight = {}

        for i in (2, 3, 0, 1):
            for g in range(3):
                q = chunk(g, 0, i, sent=True)
                acc[pl.ds(q * ch, ch), pl.ds(col0s[g], widths[g])] = partial(q, g)
                rdma = make_rdma(g, 0, i)
                rdma.start()
                inflight[(g, 0, i)] = rdma
        for i in range(4):
            for g in range(3):
                q = chunk(g, 0, i, sent=False)
                acc[pl.ds(q * ch, ch), pl.ds(col0s[g], widths[g])] = partial(q, g)

        def fold(g, k, i):
            slot = SLOT_BASE[k] + i
            inflight[(g, k, i)].wait_recv()
            q = chunk(g, k, i, sent=False)
            rows = pl.ds(q * ch, ch)
            cols = pl.ds(col0s[g], widths[g])
            summed = (acc[rows, cols].astype(jnp.float32)
                      + recvs[g][slot].astype(jnp.float32))
            if k < 2:
                acc[rows, cols] = summed.astype(jnp.bfloat16)
            else:
                out_ref[:, cols] = summed

        for g in range(3):
            for i in (2, 3):
                fold(g, 0, i)
            for i2 in (1, 0):
                rdma = make_rdma(g, 1, i2)
                rdma.start()
                inflight[(g, 1, i2)] = rdma
        for g in range(3):
            for i in (0, 1):
                fold(g, 0, i)
        for g in range(3):
            fold(g, 1, 1)
            rdma = make_rdma(g, 2, 0)
            rdma.start()
            inflight[(g, 2, 0)] = rdma
        for g in range(3):
            fold(g, 1, 0)
        for g in range(3):
            fold(g, 2, 0)

        for k in range(3):
            for g in range(3):
                for i in range(4 >> k):
                    inflight[(g, k, i)].wait_send()

    return pl.pallas_call(
        body,
        out_shape=jax.ShapeDtypeStruct((ch, f), jnp.float32),
        in_specs=[
            pl.BlockSpec(memory_space=pltpu.VMEM),
            pl.BlockSpec(memory_space=pltpu.VMEM),
        ],
        out_specs=pl.BlockSpec(memory_space=pltpu.VMEM),
        scratch_shapes=[
            pltpu.VMEM((d, f), jnp.bfloat16),
            pltpu.VMEM((7, ch, widths[0]), jnp.bfloat16),
            pltpu.VMEM((7, ch, widths[1]), jnp.bfloat16),
            pltpu.VMEM((7, ch, widths[2]), jnp.bfloat16),
            pltpu.SemaphoreType.DMA((7,)),
            pltpu.SemaphoreType.DMA((7,)),
            pltpu.SemaphoreType.DMA((7,)),
            pltpu.SemaphoreType.DMA((7,)),
            pltpu.SemaphoreType.DMA((7,)),
            pltpu.SemaphoreType.DMA((7,)),
        ],
        compiler_params=pltpu.CompilerParams(collective_id=0),
    )(x_bf, dy_bf)


# device time: 44161 ns/iter; 1.2615x vs baseline; 1.0014x over previous
import jax
import jax.numpy as jnp
from jax import lax
from jax.experimental import pallas as pl
from jax.experimental.pallas import tpu as pltpu

N_DEV = 8
ORD = ((0, 1, 2), (1, 2, 0), (2, 0, 1))
SLOT_BASE = (0, 4, 6)


def kernel(x, dy):
    m, d = x.shape
    _, f = dy.shape
    ch = d // N_DEV
    widths = (11 * ch, 11 * ch, 10 * ch)
    col0s = (0, widths[0], widths[0] + widths[1])

    x_bf = x.astype(jnp.bfloat16)
    dy_bf = dy.astype(jnp.bfloat16)

    def body(x_ref, dy_ref, out_ref, acc,
             recv0, recv1, recv2, ss0, rs0, ss1, rs1, ss2, rs2):
        recvs = (recv0, recv1, recv2)
        sss = (ss0, ss1, ss2)
        rss = (rs0, rs1, rs2)

        j = lax.axis_index("i")
        jj = j % 4
        c = [((jj == 1) | (jj == 2)).astype(jnp.int32),
             (jj >= 2).astype(jnp.int32),
             j // 4]

        def qid(qx, qy, qz):
            return qz * 4 + qy * 2 + (qx + qy) % 2

        partners = []
        for dd in range(3):
            p = list(c)
            p[dd] = 1 - p[dd]
            partners.append(qid(p[0], p[1], p[2]))

        def chunk(g, k, i, sent):
            d0, d1, d2 = ORD[g]

            def rel(dd, r):
                return c[dd] if r == 0 else 1 - c[dd]

            bits = [None, None, None]
            if k == 0:
                bits[d0] = rel(d0, 1 if sent else 0)
                bits[d1] = rel(d1, (i >> 1) & 1)
                bits[d2] = rel(d2, i & 1)
            elif k == 1:
                bits[d0] = rel(d0, 0)
                bits[d1] = rel(d1, 1 if sent else 0)
                bits[d2] = rel(d2, i)
            else:
                bits[d0] = rel(d0, 0)
                bits[d1] = rel(d1, 0)
                bits[d2] = rel(d2, 1 if sent else 0)
            return qid(bits[0], bits[1], bits[2])

        barrier = pltpu.get_barrier_semaphore()
        for dd in range(3):
            pl.semaphore_signal(barrier, inc=1, device_id=(partners[dd],),
                                device_id_type=pl.DeviceIdType.MESH)
        pl.semaphore_wait(barrier, 3)

        def partial(q, g):
            return lax.dot_general(
                x_ref[:, pl.ds(q * ch, ch)],
                dy_ref[:, pl.ds(col0s[g], widths[g])],
                (((0,), (0,)), ((), ())),
                preferred_element_type=jnp.float32,
            ).astype(jnp.bfloat16)

        def make_rdma(g, k, i):
            q = chunk(g, k, i, sent=True)
            slot = SLOT_BASE[k] + i
            return pltpu.make_async_remote_copy(
                src_ref=acc.at[pl.ds(q * ch, ch),
                               pl.ds(col0s[g], widths[g])],
                dst_ref=recvs[g].at[slot],
                send_sem=sss[g].at[slot],
                recv_sem=rss[g].at[slot],
                device_id=(partners[ORD[g][k]],),
                device_id_type=pl.DeviceIdType.MESH,
            )

        inflight = {}

        for i in (3, 2, 0, 1):
            for g in range(3):
                q = chunk(g, 0, i, sent=True)
                acc[pl.ds(q * ch, ch), pl.ds(col0s[g], widths[g])] = partial(q, g)
                rdma = make_rdma(g, 0, i)
                rdma.start()
                inflight[(g, 0, i)] = rdma
        for i in range(4):
            for g in range(3):
                q = chunk(g, 0, i, sent=False)
                acc[pl.ds(q * ch, ch), pl.ds(col0s[g], widths[g])] = partial(q, g)

        def fold(g, k, i):
            slot = SLOT_BASE[k] + i
            inflight[(g, k, i)].wait_recv()
            q = chunk(g, k, i, sent=False)
            rows = pl.ds(q * ch, ch)
            cols = pl.ds(col0s[g], widths[g])
            summed = (acc[rows, cols].astype(jnp.float32)
                      + recvs[g][slot].astype(jnp.float32))
            if k < 2:
                acc[rows, cols] = summed.astype(jnp.bfloat16)
            else:
                out_ref[:, cols] = summed

        for g in range(3):
            for i2 in (1, 0):
                fold(g, 0, 2 + i2)
                rdma = make_rdma(g, 1, i2)
                rdma.start()
                inflight[(g, 1, i2)] = rdma
        for g in range(3):
            for i in (0, 1):
                fold(g, 0, i)
        for g in range(3):
            fold(g, 1, 1)
            rdma = make_rdma(g, 2, 0)
            rdma.start()
            inflight[(g, 2, 0)] = rdma
        for g in range(3):
            fold(g, 1, 0)
        for g in range(3):
            fold(g, 2, 0)

        for k in range(3):
            for g in range(3):
                for i in range(4 >> k):
                    inflight[(g, k, i)].wait_send()

    return pl.pallas_call(
        body,
        out_shape=jax.ShapeDtypeStruct((ch, f), jnp.float32),
        in_specs=[
            pl.BlockSpec(memory_space=pltpu.VMEM),
            pl.BlockSpec(memory_space=pltpu.VMEM),
        ],
        out_specs=pl.BlockSpec(memory_space=pltpu.VMEM),
        scratch_shapes=[
            pltpu.VMEM((d, f), jnp.bfloat16),
            pltpu.VMEM((7, ch, widths[0]), jnp.bfloat16),
            pltpu.VMEM((7, ch, widths[1]), jnp.bfloat16),
            pltpu.VMEM((7, ch, widths[2]), jnp.bfloat16),
            pltpu.SemaphoreType.DMA((7,)),
            pltpu.SemaphoreType.DMA((7,)),
            pltpu.SemaphoreType.DMA((7,)),
            pltpu.SemaphoreType.DMA((7,)),
            pltpu.SemaphoreType.DMA((7,)),
            pltpu.SemaphoreType.DMA((7,)),
        ],
        compiler_params=pltpu.CompilerParams(collective_id=0),
    )(x_bf, dy_bf)
